# Initial kernel scaffold; baseline (speedup 1.0000x reference)
import jax
import jax.numpy as jnp
from jax.experimental import pallas as pl


def kernel(exercise_seq, concept_seq, response_seq, exercise_table, concept_table, difficult_table, a_table, W, b):
    raise NotImplementedError("write your pallas kernel here")


# speedup vs baseline: 8.2373x; 8.2373x over previous
import functools

import jax
import jax.numpy as jnp
from jax import lax
from jax.experimental import pallas as pl
from jax.experimental.pallas import tpu as pltpu
from jax.experimental.pallas import tpu_sc as plsc

B, S, D = 4096, 200, 128
N = B * S
NC, NS = 2, 16
NW = NC * NS
PER_W = N // NW
C = 128
CHUNKS = PER_W // C


def _sc_body(e_idx, c_idx, ex_t, con_t, diff_t, q_out, pid_out,
             eidx_v, cidx_v, exb, conb, pidb, sem_e, sem_c, sem_p):
    wid = lax.axis_index("s") * NC + lax.axis_index("c")

    def chunk(i, _):
        base = wid * PER_W + i * C
        pltpu.sync_copy(e_idx.at[pl.ds(base, C)], eidx_v)
        pltpu.sync_copy(c_idx.at[pl.ds(base, C)], cidx_v)
        de = pltpu.async_copy(ex_t.at[eidx_v], exb, sem_e)
        dc = pltpu.async_copy(con_t.at[cidx_v], conb, sem_c)
        dp = pltpu.async_copy(diff_t.at[eidx_v], pidb, sem_p)
        de.wait()
        dc.wait()
        dp.wait()

        def group(g, _):
            pidv16 = pidb[pl.ds(g * 16, 16)]
            for k in range(16):
                t = g * 16 + k
                pv = pidv16[k]
                for j in range(D // 16):
                    sl = (t, pl.ds(j * 16, 16))
                    conb[sl] = conb[sl] + pv * exb[sl]
            return 0

        lax.fori_loop(0, C // 16, group, 0)
        pltpu.sync_copy(conb, q_out.at[pl.ds(base, C)])
        pltpu.sync_copy(pidb, pid_out.at[pl.ds(base, C)])
        return 0

    lax.fori_loop(0, CHUNKS, chunk, 0)


def _sc_gather_combine(e_flat, c_flat, ex_t, con_t, diff_flat):
    mesh = plsc.VectorSubcoreMesh(core_axis_name="c", subcore_axis_name="s",
                                  num_cores=NC, num_subcores=NS)
    f = pl.kernel(
        _sc_body,
        out_type=[jax.ShapeDtypeStruct((N, D), jnp.float32),
                  jax.ShapeDtypeStruct((N,), jnp.float32)],
        mesh=mesh,
        scratch_types=[
            pltpu.VMEM((C,), jnp.int32),
            pltpu.VMEM((C,), jnp.int32),
            pltpu.VMEM((C, D), jnp.float32),
            pltpu.VMEM((C, D), jnp.float32),
            pltpu.VMEM((C,), jnp.float32),
            pltpu.SemaphoreType.DMA,
            pltpu.SemaphoreType.DMA,
            pltpu.SemaphoreType.DMA,
        ],
    )
    return f(e_flat, c_flat, ex_t, con_t, diff_flat)


R = 1024


def _tc_body(q_ref, m_ref, w1t_ref, row0_ref, diff_ref, out_ref):
    acc = jax.lax.dot_general(
        q_ref[...], w1t_ref[...], (((1,), (0,)), ((), ())),
        preferred_element_type=jnp.float32,
        precision=jax.lax.Precision.HIGHEST)
    out_ref[...] = acc + row0_ref[...] + m_ref[...] * diff_ref[...]


def _tc_linear(q, respf, w1t, row0, diff):
    grid = (N // R,)
    return pl.pallas_call(
        _tc_body,
        grid=grid,
        in_specs=[
            pl.BlockSpec((R, D), lambda i: (i, 0)),
            pl.BlockSpec((R, 1), lambda i: (i, 0)),
            pl.BlockSpec((D, D), lambda i: (0, 0)),
            pl.BlockSpec((1, D), lambda i: (0, 0)),
            pl.BlockSpec((1, D), lambda i: (0, 0)),
        ],
        out_specs=pl.BlockSpec((R, D), lambda i: (i, 0)),
        out_shape=jax.ShapeDtypeStruct((N, D), jnp.float32),
    )(q, respf, w1t, row0, diff)


def kernel(exercise_seq, concept_seq, response_seq, exercise_table,
           concept_table, difficult_table, a_table, W, b):
    e_flat = exercise_seq.reshape(-1).astype(jnp.int32)
    c_flat = concept_seq.reshape(-1).astype(jnp.int32)
    diff_flat = difficult_table.reshape(-1)

    q_flat, pid_flat = _sc_gather_combine(
        e_flat, c_flat, exercise_table, concept_table, diff_flat)

    w1t = W[:, :D].T
    w2t = W[:, D:].T
    rows = a_table @ w2t + b[None, :]
    row0 = rows[0:1, :]
    diff = rows[1:2, :] - row0
    respf = response_seq.reshape(-1, 1).astype(jnp.float32)

    qa_flat = _tc_linear(q_flat, respf, w1t, row0, diff)

    q = q_flat.reshape(B, S, D)
    qa = qa_flat.reshape(B, S, D)
    pid = pid_flat.reshape(B, S, 1)
    return (q, qa, pid)


# speedup vs baseline: 10.9608x; 1.3306x over previous
import functools

import jax
import jax.numpy as jnp
from jax import lax
from jax.experimental import pallas as pl
from jax.experimental.pallas import tpu as pltpu
from jax.experimental.pallas import tpu_sc as plsc

B, S, D = 4096, 200, 128
N = B * S
NC, NS = 2, 16
NW = NC * NS
PER_W = N // NW
C = 128
CHUNKS = PER_W // C
CON_ROWS = 1001


def _sc_body(e_idx, c_idx, ex_t, con_t, diff_t, q_out, pid_out,
             eidx_v, cidx_v, exb, conb, pidb, con_sh,
             sem_e, sem_c, sem_p, sem_wb):
    cid = lax.axis_index("c")
    sid = lax.axis_index("s")
    wid = sid * NC + cid
    base_w = wid * PER_W

    @pl.when(sid == 0)
    def _():
        pltpu.sync_copy(con_t, con_sh)

    pltpu.sync_copy(e_idx.at[pl.ds(base_w, PER_W)], eidx_v)
    pltpu.sync_copy(c_idx.at[pl.ds(base_w, PER_W)], cidx_v)
    plsc.subcore_barrier()

    def fire_gathers(i, b):
        off = i * C
        es = eidx_v.at[pl.ds(off, C)]
        cs = cidx_v.at[pl.ds(off, C)]
        pltpu.async_copy(ex_t.at[es], exb.at[b], sem_e.at[b])
        pltpu.async_copy(con_sh.at[cs], conb.at[b], sem_c.at[b])
        pltpu.async_copy(diff_t.at[es], pidb.at[b], sem_p.at[b])

    def wait_gathers(i, b):
        off = i * C
        es = eidx_v.at[pl.ds(off, C)]
        cs = cidx_v.at[pl.ds(off, C)]
        pltpu.make_async_copy(ex_t.at[es], exb.at[b], sem_e.at[b]).wait()
        pltpu.make_async_copy(con_sh.at[cs], conb.at[b], sem_c.at[b]).wait()
        pltpu.make_async_copy(diff_t.at[es], pidb.at[b], sem_p.at[b]).wait()

    def fire_wb(i, b):
        base = base_w + i * C
        pltpu.async_copy(conb.at[b], q_out.at[pl.ds(base, C)], sem_wb.at[b])
        pltpu.async_copy(pidb.at[b], pid_out.at[pl.ds(base, C)], sem_wb.at[b])

    def wait_wb(b):
        pltpu.make_async_copy(conb.at[b], q_out.at[pl.ds(base_w, C)],
                              sem_wb.at[b]).wait()
        pltpu.make_async_copy(pidb.at[b], pid_out.at[pl.ds(base_w, C)],
                              sem_wb.at[b]).wait()

    def combine(b):
        def group(g, _):
            pidv16 = pidb[b, pl.ds(g * 16, 16)]
            for k in range(16):
                t = g * 16 + k
                pv = pidv16[k]
                for j in range(D // 16):
                    sl = (b, t, pl.ds(j * 16, 16))
                    conb[sl] = conb[sl] + pv * exb[sl]
            return 0

        lax.fori_loop(0, C // 16, group, 0)

    fire_gathers(0, 0)

    def pair(p, _):
        for b in range(2):
            i = 2 * p + b
            wait_gathers(i, b)
            if b == 0:
                @pl.when(p > 0)
                def _():
                    wait_wb(1)
                fire_gathers(i + 1, 1)
            else:
                wait_wb(0)

                @pl.when(p < CHUNKS // 2 - 1)
                def _():
                    fire_gathers(i + 1, 0)
            combine(b)
            fire_wb(i, b)
        return 0

    lax.fori_loop(0, CHUNKS // 2, pair, 0)
    wait_wb(1)


def _sc_gather_combine(e_flat, c_flat, ex_t, con_t, diff_flat):
    mesh = plsc.VectorSubcoreMesh(core_axis_name="c", subcore_axis_name="s",
                                  num_cores=NC, num_subcores=NS)
    f = pl.kernel(
        _sc_body,
        out_type=[jax.ShapeDtypeStruct((N, D), jnp.float32),
                  jax.ShapeDtypeStruct((N,), jnp.float32)],
        mesh=mesh,
        scratch_types=[
            pltpu.VMEM((PER_W,), jnp.int32),
            pltpu.VMEM((PER_W,), jnp.int32),
            pltpu.VMEM((2, C, D), jnp.float32),
            pltpu.VMEM((2, C, D), jnp.float32),
            pltpu.VMEM((2, C), jnp.float32),
            pltpu.VMEM_SHARED((CON_ROWS, D), jnp.float32),
            pltpu.SemaphoreType.DMA((2,)),
            pltpu.SemaphoreType.DMA((2,)),
            pltpu.SemaphoreType.DMA((2,)),
            pltpu.SemaphoreType.DMA((2,)),
        ],
    )
    return f(e_flat, c_flat, ex_t, con_t, diff_flat)


R = 1024


def _tc_body(q_ref, m_ref, w1t_ref, row0_ref, diff_ref, out_ref):
    acc = jax.lax.dot_general(
        q_ref[...], w1t_ref[...], (((1,), (0,)), ((), ())),
        preferred_element_type=jnp.float32,
        precision=jax.lax.Precision.HIGHEST)
    out_ref[...] = acc + row0_ref[...] + m_ref[...] * diff_ref[...]


def _tc_linear(q, respf, w1t, row0, diff):
    grid = (N // R,)
    return pl.pallas_call(
        _tc_body,
        grid=grid,
        in_specs=[
            pl.BlockSpec((R, D), lambda i: (i, 0)),
            pl.BlockSpec((R, 1), lambda i: (i, 0)),
            pl.BlockSpec((D, D), lambda i: (0, 0)),
            pl.BlockSpec((1, D), lambda i: (0, 0)),
            pl.BlockSpec((1, D), lambda i: (0, 0)),
        ],
        out_specs=pl.BlockSpec((R, D), lambda i: (i, 0)),
        out_shape=jax.ShapeDtypeStruct((N, D), jnp.float32),
    )(q, respf, w1t, row0, diff)


def kernel(exercise_seq, concept_seq, response_seq, exercise_table,
           concept_table, difficult_table, a_table, W, b):
    e_flat = exercise_seq.reshape(-1).astype(jnp.int32)
    c_flat = concept_seq.reshape(-1).astype(jnp.int32)
    diff_flat = difficult_table.reshape(-1)

    q_flat, pid_flat = _sc_gather_combine(
        e_flat, c_flat, exercise_table, concept_table, diff_flat)

    w1t = W[:, :D].T
    w2t = W[:, D:].T
    rows = a_table @ w2t + b[None, :]
    row0 = rows[0:1, :]
    diff = rows[1:2, :] - row0
    respf = response_seq.reshape(-1, 1).astype(jnp.float32)

    qa_flat = _tc_linear(q_flat, respf, w1t, row0, diff)

    q = q_flat.reshape(B, S, D)
    qa = qa_flat.reshape(B, S, D)
    pid = pid_flat.reshape(B, S, 1)
    return (q, qa, pid)


# speedup vs baseline: 12.5704x; 1.1468x over previous
import functools

import jax
import jax.numpy as jnp
from jax import lax
from jax.experimental import pallas as pl
from jax.experimental.pallas import tpu as pltpu
from jax.experimental.pallas import tpu_sc as plsc

B, S, D = 4096, 200, 128
N = B * S
NC, NS = 2, 16
NW = NC * NS
PER_W = N // NW
C = 128
CHUNKS = PER_W // C
CON_ROWS = 1001


def _sc_body(e_idx, c_idx, ex_t, con_t, diff_t, q_out, pid_out,
             eidx_v, cidx_v, exb, conb, pidb, con_sh,
             sem_e, sem_c, sem_p, sem_wb):
    cid = lax.axis_index("c")
    sid = lax.axis_index("s")
    wid = sid * NC + cid
    base_w = wid * PER_W

    @pl.when(sid == 0)
    def _():
        pltpu.sync_copy(con_t, con_sh)

    pltpu.sync_copy(e_idx.at[pl.ds(base_w, PER_W)], eidx_v)
    pltpu.sync_copy(c_idx.at[pl.ds(base_w, PER_W)], cidx_v)
    plsc.subcore_barrier()

    def fire_gathers(i, b):
        off = i * C
        es = eidx_v.at[pl.ds(off, C)]
        cs = cidx_v.at[pl.ds(off, C)]
        pltpu.async_copy(ex_t.at[es], exb.at[b], sem_e.at[b])
        pltpu.async_copy(con_sh.at[cs], conb.at[b], sem_c.at[b])
        pltpu.async_copy(diff_t.at[es], pidb.at[b], sem_p.at[b])

    def wait_gathers(i, b):
        off = i * C
        es = eidx_v.at[pl.ds(off, C)]
        cs = cidx_v.at[pl.ds(off, C)]
        pltpu.make_async_copy(ex_t.at[es], exb.at[b], sem_e.at[b]).wait()
        pltpu.make_async_copy(con_sh.at[cs], conb.at[b], sem_c.at[b]).wait()
        pltpu.make_async_copy(diff_t.at[es], pidb.at[b], sem_p.at[b]).wait()

    def fire_wb(i, b):
        base = base_w + i * C
        pltpu.async_copy(conb.at[b], q_out.at[pl.ds(base, C)], sem_wb.at[b])
        pltpu.async_copy(pidb.at[b], pid_out.at[pl.ds(base, C)], sem_wb.at[b])

    def wait_wb(b):
        pltpu.make_async_copy(conb.at[b], q_out.at[pl.ds(base_w, C)],
                              sem_wb.at[b]).wait()
        pltpu.make_async_copy(pidb.at[b], pid_out.at[pl.ds(base_w, C)],
                              sem_wb.at[b]).wait()

    def combine(b):
        def group(g, _):
            pidv16 = pidb[b, pl.ds(g * 16, 16)]
            for k in range(16):
                t = g * 16 + k
                pv = pidv16[k]
                for j in range(D // 16):
                    sl = (b, t, pl.ds(j * 16, 16))
                    conb[sl] = conb[sl] + pv * exb[sl]
            return 0

        lax.fori_loop(0, C // 16, group, 0)

    fire_gathers(0, 0)

    def pair(p, _):
        for b in range(2):
            i = 2 * p + b
            wait_gathers(i, b)
            if b == 0:
                @pl.when(p > 0)
                def _():
                    wait_wb(1)
                fire_gathers(i + 1, 1)
            else:
                wait_wb(0)

                @pl.when(p < CHUNKS // 2 - 1)
                def _():
                    fire_gathers(i + 1, 0)
            combine(b)
            fire_wb(i, b)
        return 0

    lax.fori_loop(0, CHUNKS // 2, pair, 0)
    wait_wb(1)


def _sc_gather_combine(e_flat, c_flat, ex_t, con_t, diff_flat):
    mesh = plsc.VectorSubcoreMesh(core_axis_name="c", subcore_axis_name="s",
                                  num_cores=NC, num_subcores=NS)
    f = pl.kernel(
        _sc_body,
        out_type=[jax.ShapeDtypeStruct((N, D), jnp.float32),
                  jax.ShapeDtypeStruct((N,), jnp.float32)],
        mesh=mesh,
        scratch_types=[
            pltpu.VMEM((PER_W,), jnp.int32),
            pltpu.VMEM((PER_W,), jnp.int32),
            pltpu.VMEM((2, C, D), jnp.float32),
            pltpu.VMEM((2, C, D), jnp.float32),
            pltpu.VMEM((2, C), jnp.float32),
            pltpu.VMEM_SHARED((CON_ROWS, D), jnp.float32),
            pltpu.SemaphoreType.DMA((2,)),
            pltpu.SemaphoreType.DMA((2,)),
            pltpu.SemaphoreType.DMA((2,)),
            pltpu.SemaphoreType.DMA((2,)),
        ],
    )
    return f(e_flat, c_flat, ex_t, con_t, diff_flat)


R = 2048


def _tc_body(q_ref, m_ref, w1t_ref, row0_ref, diff_ref, out_ref):
    acc = jax.lax.dot_general(
        q_ref[...], w1t_ref[...], (((1,), (0,)), ((), ())),
        preferred_element_type=jnp.float32,
        precision=jax.lax.Precision.HIGHEST)
    out_ref[...] = acc + row0_ref[...] + m_ref[...] * diff_ref[...]


def _tc_linear(q, respf, w1t, row0, diff):
    grid = (N // R,)
    return pl.pallas_call(
        _tc_body,
        grid=grid,
        in_specs=[
            pl.BlockSpec((R, D), lambda i: (i, 0)),
            pl.BlockSpec((R, 1), lambda i: (i, 0)),
            pl.BlockSpec((D, D), lambda i: (0, 0)),
            pl.BlockSpec((1, D), lambda i: (0, 0)),
            pl.BlockSpec((1, D), lambda i: (0, 0)),
        ],
        out_specs=pl.BlockSpec((R, D), lambda i: (i, 0)),
        out_shape=jax.ShapeDtypeStruct((N, D), jnp.float32),
    )(q, respf, w1t, row0, diff)


def kernel(exercise_seq, concept_seq, response_seq, exercise_table,
           concept_table, difficult_table, a_table, W, b):
    e_flat = exercise_seq.reshape(-1).astype(jnp.int32)
    c_flat = concept_seq.reshape(-1).astype(jnp.int32)
    diff_flat = difficult_table.reshape(-1)

    q_flat, pid_flat = _sc_gather_combine(
        e_flat, c_flat, exercise_table, concept_table, diff_flat)

    w1t = W[:, :D].T
    w2t = W[:, D:].T
    rows = a_table @ w2t + b[None, :]
    row0 = rows[0:1, :]
    diff = rows[1:2, :] - row0
    respf = response_seq.reshape(-1, 1).astype(jnp.float32)

    qa_flat = _tc_linear(q_flat, respf, w1t, row0, diff)

    q = q_flat.reshape(B, S, D)
    qa = qa_flat.reshape(B, S, D)
    pid = pid_flat.reshape(B, S, 1)
    return (q, qa, pid)


# speedup vs baseline: 13.7536x; 1.0941x over previous
import functools

import jax
import jax.numpy as jnp
from jax import lax
from jax.experimental import pallas as pl
from jax.experimental.pallas import tpu as pltpu
from jax.experimental.pallas import tpu_sc as plsc

B, S, D = 4096, 200, 128
N = B * S
NC, NS = 2, 16
NW = NC * NS
PER_W = N // NW
C = 128
CHUNKS = PER_W // C
CON_ROWS = 1001


def _sc_body(e_idx, c_idx, ex_t, con_t, diff_t, q_out, pid_out,
             eidx_v, cidx_v, exb, conb, pidb, con_sh,
             sem_e, sem_c, sem_p, sem_wb):
    cid = lax.axis_index("c")
    sid = lax.axis_index("s")
    wid = sid * NC + cid
    base_w = wid * PER_W

    @pl.when(sid == 0)
    def _():
        pltpu.sync_copy(con_t, con_sh)

    pltpu.sync_copy(e_idx.at[pl.ds(base_w, PER_W)], eidx_v)
    pltpu.sync_copy(c_idx.at[pl.ds(base_w, PER_W)], cidx_v)
    plsc.subcore_barrier()

    def fire_gathers(i, b):
        off = i * C
        es = eidx_v.at[pl.ds(off, C)]
        cs = cidx_v.at[pl.ds(off, C)]
        pltpu.async_copy(ex_t.at[es], exb.at[b], sem_e.at[b])
        pltpu.async_copy(con_sh.at[cs], conb.at[b], sem_c.at[b])
        pltpu.async_copy(diff_t.at[es], pidb.at[b], sem_p.at[b])

    def wait_gathers(i, b):
        off = i * C
        es = eidx_v.at[pl.ds(off, C)]
        cs = cidx_v.at[pl.ds(off, C)]
        pltpu.make_async_copy(ex_t.at[es], exb.at[b], sem_e.at[b]).wait()
        pltpu.make_async_copy(con_sh.at[cs], conb.at[b], sem_c.at[b]).wait()
        pltpu.make_async_copy(diff_t.at[es], pidb.at[b], sem_p.at[b]).wait()

    def fire_wb(i, b):
        base = base_w + i * C
        pltpu.async_copy(conb.at[b], q_out.at[pl.ds(base, C)], sem_wb.at[b])
        pltpu.async_copy(pidb.at[b], pid_out.at[pl.ds(base, C)], sem_wb.at[b])

    def wait_wb(b):
        pltpu.make_async_copy(conb.at[b], q_out.at[pl.ds(base_w, C)],
                              sem_wb.at[b]).wait()
        pltpu.make_async_copy(pidb.at[b], pid_out.at[pl.ds(base_w, C)],
                              sem_wb.at[b]).wait()

    def combine(b):
        def group(g, _):
            pidv16 = pidb[b, pl.ds(g * 16, 16)]
            for k in range(16):
                t = g * 16 + k
                pv = pidv16[k]
                for j in range(D // 16):
                    sl = (b, t, pl.ds(j * 16, 16))
                    conb[sl] = conb[sl] + pv * exb[sl]
            return 0

        lax.fori_loop(0, C // 16, group, 0)

    fire_gathers(0, 0)

    def pair(p, _):
        for b in range(2):
            i = 2 * p + b
            wait_gathers(i, b)
            if b == 0:
                @pl.when(p > 0)
                def _():
                    wait_wb(1)
                fire_gathers(i + 1, 1)
            else:
                wait_wb(0)

                @pl.when(p < CHUNKS // 2 - 1)
                def _():
                    fire_gathers(i + 1, 0)
            combine(b)
            fire_wb(i, b)
        return 0

    lax.fori_loop(0, CHUNKS // 2, pair, 0)
    wait_wb(1)


def _sc_gather_combine(e_flat, c_flat, ex_t, con_t, diff_flat):
    mesh = plsc.VectorSubcoreMesh(core_axis_name="c", subcore_axis_name="s",
                                  num_cores=NC, num_subcores=NS)
    f = pl.kernel(
        _sc_body,
        out_type=[jax.ShapeDtypeStruct((N, D), jnp.float32),
                  jax.ShapeDtypeStruct((N,), jnp.float32)],
        mesh=mesh,
        scratch_types=[
            pltpu.VMEM((PER_W,), jnp.int32),
            pltpu.VMEM((PER_W,), jnp.int32),
            pltpu.VMEM((2, C, D), jnp.float32),
            pltpu.VMEM((2, C, D), jnp.float32),
            pltpu.VMEM((2, C), jnp.float32),
            pltpu.VMEM_SHARED((CON_ROWS, D), jnp.float32),
            pltpu.SemaphoreType.DMA((2,)),
            pltpu.SemaphoreType.DMA((2,)),
            pltpu.SemaphoreType.DMA((2,)),
            pltpu.SemaphoreType.DMA((2,)),
        ],
    )
    return f(e_flat, c_flat, ex_t, con_t, diff_flat)


R = 4096


def _tc_body(q_ref, m_ref, w1t_ref, row0_ref, diff_ref, out_ref):
    acc = jax.lax.dot_general(
        q_ref[...], w1t_ref[...], (((1,), (0,)), ((), ())),
        preferred_element_type=jnp.float32,
        precision=jax.lax.Precision.HIGHEST)
    out_ref[...] = acc + row0_ref[...] + m_ref[...] * diff_ref[...]


def _tc_linear(q, respf, w1t, row0, diff):
    grid = (N // R,)
    return pl.pallas_call(
        _tc_body,
        grid=grid,
        in_specs=[
            pl.BlockSpec((R, D), lambda i: (i, 0)),
            pl.BlockSpec((R, 1), lambda i: (i, 0)),
            pl.BlockSpec((D, D), lambda i: (0, 0)),
            pl.BlockSpec((1, D), lambda i: (0, 0)),
            pl.BlockSpec((1, D), lambda i: (0, 0)),
        ],
        out_specs=pl.BlockSpec((R, D), lambda i: (i, 0)),
        out_shape=jax.ShapeDtypeStruct((N, D), jnp.float32),
    )(q, respf, w1t, row0, diff)


def kernel(exercise_seq, concept_seq, response_seq, exercise_table,
           concept_table, difficult_table, a_table, W, b):
    e_flat = exercise_seq.reshape(-1).astype(jnp.int32)
    c_flat = concept_seq.reshape(-1).astype(jnp.int32)
    diff_flat = difficult_table.reshape(-1)

    q_flat, pid_flat = _sc_gather_combine(
        e_flat, c_flat, exercise_table, concept_table, diff_flat)

    w1t = W[:, :D].T
    w2t = W[:, D:].T
    rows = a_table @ w2t + b[None, :]
    row0 = rows[0:1, :]
    diff = rows[1:2, :] - row0
    respf = response_seq.reshape(-1, 1).astype(jnp.float32)

    qa_flat = _tc_linear(q_flat, respf, w1t, row0, diff)

    q = q_flat.reshape(B, S, D)
    qa = qa_flat.reshape(B, S, D)
    pid = pid_flat.reshape(B, S, 1)
    return (q, qa, pid)


# speedup vs baseline: 14.4553x; 1.0510x over previous
import functools

import jax
import jax.numpy as jnp
from jax import lax
from jax.experimental import pallas as pl
from jax.experimental.pallas import tpu as pltpu
from jax.experimental.pallas import tpu_sc as plsc

B, S, D = 4096, 200, 128
N = B * S
NC, NS = 2, 16
NW = NC * NS
PER_W = N // NW
C = 128
CHUNKS = PER_W // C
CON_ROWS = 1001


def _sc_body(e_idx, c_idx, ex_t, con_t, diff_t, q_out, pid_out,
             eidx_v, cidx_v, exb, conb, pidb, con_sh,
             sem_e, sem_c, sem_p, sem_wb):
    cid = lax.axis_index("c")
    sid = lax.axis_index("s")
    wid = sid * NC + cid
    base_w = wid * PER_W

    @pl.when(sid == 0)
    def _():
        pltpu.sync_copy(con_t, con_sh)

    pltpu.sync_copy(e_idx.at[pl.ds(base_w, PER_W)], eidx_v)
    pltpu.sync_copy(c_idx.at[pl.ds(base_w, PER_W)], cidx_v)
    plsc.subcore_barrier()

    def fire_gathers(i, b):
        off = i * C
        es = eidx_v.at[pl.ds(off, C)]
        cs = cidx_v.at[pl.ds(off, C)]
        pltpu.async_copy(ex_t.at[es], exb.at[b], sem_e.at[b])
        pltpu.async_copy(con_sh.at[cs], conb.at[b], sem_c.at[b])
        pltpu.async_copy(diff_t.at[es], pidb.at[b], sem_p.at[b])

    def wait_gathers(i, b):
        off = i * C
        es = eidx_v.at[pl.ds(off, C)]
        cs = cidx_v.at[pl.ds(off, C)]
        pltpu.make_async_copy(ex_t.at[es], exb.at[b], sem_e.at[b]).wait()
        pltpu.make_async_copy(con_sh.at[cs], conb.at[b], sem_c.at[b]).wait()
        pltpu.make_async_copy(diff_t.at[es], pidb.at[b], sem_p.at[b]).wait()

    def fire_wb(i, b):
        base = base_w + i * C
        pltpu.async_copy(conb.at[b], q_out.at[pl.ds(base, C)], sem_wb.at[b])
        pltpu.async_copy(pidb.at[b], pid_out.at[pl.ds(base, C)], sem_wb.at[b])

    def wait_wb(b):
        pltpu.make_async_copy(conb.at[b], q_out.at[pl.ds(base_w, C)],
                              sem_wb.at[b]).wait()
        pltpu.make_async_copy(pidb.at[b], pid_out.at[pl.ds(base_w, C)],
                              sem_wb.at[b]).wait()

    def combine(b):
        def group(g, _):
            pidv16 = pidb[b, pl.ds(g * 16, 16)]
            for k in range(16):
                t = g * 16 + k
                pv = pidv16[k]
                for j in range(D // 16):
                    sl = (b, t, pl.ds(j * 16, 16))
                    conb[sl] = conb[sl] + pv * exb[sl]
            return 0

        lax.fori_loop(0, C // 16, group, 0)

    fire_gathers(0, 0)

    def pair(p, _):
        for b in range(2):
            i = 2 * p + b
            wait_gathers(i, b)
            if b == 0:
                @pl.when(p > 0)
                def _():
                    wait_wb(1)
                fire_gathers(i + 1, 1)
            else:
                wait_wb(0)

                @pl.when(p < CHUNKS // 2 - 1)
                def _():
                    fire_gathers(i + 1, 0)
            combine(b)
            fire_wb(i, b)
        return 0

    lax.fori_loop(0, CHUNKS // 2, pair, 0)
    wait_wb(1)


def _sc_gather_combine(e_flat, c_flat, ex_t, con_t, diff_flat):
    mesh = plsc.VectorSubcoreMesh(core_axis_name="c", subcore_axis_name="s",
                                  num_cores=NC, num_subcores=NS)
    f = pl.kernel(
        _sc_body,
        out_type=[jax.ShapeDtypeStruct((N, D), jnp.float32),
                  jax.ShapeDtypeStruct((N,), jnp.float32)],
        mesh=mesh,
        scratch_types=[
            pltpu.VMEM((PER_W,), jnp.int32),
            pltpu.VMEM((PER_W,), jnp.int32),
            pltpu.VMEM((2, C, D), jnp.float32),
            pltpu.VMEM((2, C, D), jnp.float32),
            pltpu.VMEM((2, C), jnp.float32),
            pltpu.VMEM_SHARED((CON_ROWS, D), jnp.float32),
            pltpu.SemaphoreType.DMA((2,)),
            pltpu.SemaphoreType.DMA((2,)),
            pltpu.SemaphoreType.DMA((2,)),
            pltpu.SemaphoreType.DMA((2,)),
        ],
    )
    return f(e_flat, c_flat, ex_t, con_t, diff_flat)


R = 8192


def _tc_body(q_ref, m_ref, w1t_ref, row0_ref, diff_ref, out_ref):
    acc = jax.lax.dot_general(
        q_ref[...], w1t_ref[...], (((1,), (0,)), ((), ())),
        preferred_element_type=jnp.float32,
        precision=jax.lax.Precision.HIGHEST)
    out_ref[...] = acc + row0_ref[...] + m_ref[...] * diff_ref[...]


def _tc_linear(q, respf, w1t, row0, diff):
    grid = (N // R,)
    return pl.pallas_call(
        _tc_body,
        grid=grid,
        in_specs=[
            pl.BlockSpec((R, D), lambda i: (i, 0)),
            pl.BlockSpec((R, 1), lambda i: (i, 0)),
            pl.BlockSpec((D, D), lambda i: (0, 0)),
            pl.BlockSpec((1, D), lambda i: (0, 0)),
            pl.BlockSpec((1, D), lambda i: (0, 0)),
        ],
        out_specs=pl.BlockSpec((R, D), lambda i: (i, 0)),
        out_shape=jax.ShapeDtypeStruct((N, D), jnp.float32),
    )(q, respf, w1t, row0, diff)


def kernel(exercise_seq, concept_seq, response_seq, exercise_table,
           concept_table, difficult_table, a_table, W, b):
    e_flat = exercise_seq.reshape(-1).astype(jnp.int32)
    c_flat = concept_seq.reshape(-1).astype(jnp.int32)
    diff_flat = difficult_table.reshape(-1)

    q_flat, pid_flat = _sc_gather_combine(
        e_flat, c_flat, exercise_table, concept_table, diff_flat)

    w1t = W[:, :D].T
    w2t = W[:, D:].T
    rows = a_table @ w2t + b[None, :]
    row0 = rows[0:1, :]
    diff = rows[1:2, :] - row0
    respf = response_seq.reshape(-1, 1).astype(jnp.float32)

    qa_flat = _tc_linear(q_flat, respf, w1t, row0, diff)

    q = q_flat.reshape(B, S, D)
    qa = qa_flat.reshape(B, S, D)
    pid = pid_flat.reshape(B, S, 1)
    return (q, qa, pid)


# speedup vs baseline: 15.1575x; 1.0486x over previous
import functools

import jax
import jax.numpy as jnp
from jax import lax
from jax.experimental import pallas as pl
from jax.experimental.pallas import tpu as pltpu
from jax.experimental.pallas import tpu_sc as plsc

B, S, D = 4096, 200, 128
N = B * S
NC, NS = 2, 16
NW = NC * NS
PER_W = N // NW
C = 128
CHUNKS = PER_W // C
CON_ROWS = 1001


def _sc_body(e_idx, c_idx, ex_t, con_t, diff_t, q_out, pid_out,
             eidx_v, cidx_v, exb, conb, pidb, con_sh,
             sem_e, sem_c, sem_p, sem_wb):
    cid = lax.axis_index("c")
    sid = lax.axis_index("s")
    wid = sid * NC + cid
    base_w = wid * PER_W

    @pl.when(sid == 0)
    def _():
        pltpu.sync_copy(con_t, con_sh)

    pltpu.sync_copy(e_idx.at[pl.ds(base_w, PER_W)], eidx_v)
    pltpu.sync_copy(c_idx.at[pl.ds(base_w, PER_W)], cidx_v)
    plsc.subcore_barrier()

    def fire_gathers(i, b):
        off = i * C
        es = eidx_v.at[pl.ds(off, C)]
        cs = cidx_v.at[pl.ds(off, C)]
        pltpu.async_copy(ex_t.at[es], exb.at[b], sem_e.at[b])
        pltpu.async_copy(con_sh.at[cs], conb.at[b], sem_c.at[b])
        pltpu.async_copy(diff_t.at[es], pidb.at[b], sem_p.at[b])

    def wait_gathers(i, b):
        off = i * C
        es = eidx_v.at[pl.ds(off, C)]
        cs = cidx_v.at[pl.ds(off, C)]
        pltpu.make_async_copy(ex_t.at[es], exb.at[b], sem_e.at[b]).wait()
        pltpu.make_async_copy(con_sh.at[cs], conb.at[b], sem_c.at[b]).wait()
        pltpu.make_async_copy(diff_t.at[es], pidb.at[b], sem_p.at[b]).wait()

    def fire_wb(i, b):
        base = base_w + i * C
        pltpu.async_copy(conb.at[b], q_out.at[pl.ds(base, C)], sem_wb.at[b])
        pltpu.async_copy(pidb.at[b], pid_out.at[pl.ds(base, C)], sem_wb.at[b])

    def wait_wb(b):
        pltpu.make_async_copy(conb.at[b], q_out.at[pl.ds(base_w, C)],
                              sem_wb.at[b]).wait()
        pltpu.make_async_copy(pidb.at[b], pid_out.at[pl.ds(base_w, C)],
                              sem_wb.at[b]).wait()

    def combine(b):
        def group(g, _):
            for k in range(16):
                t = g * 16 + k
                for j in range(D // 16):
                    sl = (b, t, pl.ds(j * 16, 16))
                    conb[sl] = conb[sl] + exb[sl]
            return 0

        lax.fori_loop(0, C // 16, group, 0)

    fire_gathers(0, 0)

    def pair(p, _):
        for b in range(2):
            i = 2 * p + b
            wait_gathers(i, b)
            if b == 0:
                @pl.when(p > 0)
                def _():
                    wait_wb(1)
                fire_gathers(i + 1, 1)
            else:
                wait_wb(0)

                @pl.when(p < CHUNKS // 2 - 1)
                def _():
                    fire_gathers(i + 1, 0)
            combine(b)
            fire_wb(i, b)
        return 0

    lax.fori_loop(0, CHUNKS // 2, pair, 0)
    wait_wb(1)


def _sc_gather_combine(e_flat, c_flat, ex_t, con_t, diff_flat):
    mesh = plsc.VectorSubcoreMesh(core_axis_name="c", subcore_axis_name="s",
                                  num_cores=NC, num_subcores=NS)
    f = pl.kernel(
        _sc_body,
        out_type=[jax.ShapeDtypeStruct((N, D), jnp.float32),
                  jax.ShapeDtypeStruct((N,), jnp.float32)],
        mesh=mesh,
        scratch_types=[
            pltpu.VMEM((PER_W,), jnp.int32),
            pltpu.VMEM((PER_W,), jnp.int32),
            pltpu.VMEM((2, C, D), jnp.float32),
            pltpu.VMEM((2, C, D), jnp.float32),
            pltpu.VMEM((2, C), jnp.float32),
            pltpu.VMEM_SHARED((CON_ROWS, D), jnp.float32),
            pltpu.SemaphoreType.DMA((2,)),
            pltpu.SemaphoreType.DMA((2,)),
            pltpu.SemaphoreType.DMA((2,)),
            pltpu.SemaphoreType.DMA((2,)),
        ],
    )
    return f(e_flat, c_flat, ex_t, con_t, diff_flat)


EX_ROWS = 100001
RT = 8192


def _scale_body(ex_ref, df_ref, out_ref):
    out_ref[...] = ex_ref[...] * df_ref[...]


def _scale_table(ex_t, diff_t):
    grid = ((EX_ROWS + RT - 1) // RT,)
    return pl.pallas_call(
        _scale_body,
        grid=grid,
        in_specs=[
            pl.BlockSpec((RT, D), lambda i: (i, 0)),
            pl.BlockSpec((RT, 1), lambda i: (i, 0)),
        ],
        out_specs=pl.BlockSpec((RT, D), lambda i: (i, 0)),
        out_shape=jax.ShapeDtypeStruct((EX_ROWS, D), jnp.float32),
    )(ex_t, diff_t)


R = 8192


def _tc_body(q_ref, m_ref, w1t_ref, row0_ref, diff_ref, out_ref):
    acc = jax.lax.dot_general(
        q_ref[...], w1t_ref[...], (((1,), (0,)), ((), ())),
        preferred_element_type=jnp.float32,
        precision=jax.lax.Precision.HIGHEST)
    out_ref[...] = acc + row0_ref[...] + m_ref[...] * diff_ref[...]


def _tc_linear(q, respf, w1t, row0, diff):
    grid = (N // R,)
    return pl.pallas_call(
        _tc_body,
        grid=grid,
        in_specs=[
            pl.BlockSpec((R, D), lambda i: (i, 0)),
            pl.BlockSpec((R, 1), lambda i: (i, 0)),
            pl.BlockSpec((D, D), lambda i: (0, 0)),
            pl.BlockSpec((1, D), lambda i: (0, 0)),
            pl.BlockSpec((1, D), lambda i: (0, 0)),
        ],
        out_specs=pl.BlockSpec((R, D), lambda i: (i, 0)),
        out_shape=jax.ShapeDtypeStruct((N, D), jnp.float32),
    )(q, respf, w1t, row0, diff)


def kernel(exercise_seq, concept_seq, response_seq, exercise_table,
           concept_table, difficult_table, a_table, W, b):
    e_flat = exercise_seq.reshape(-1).astype(jnp.int32)
    c_flat = concept_seq.reshape(-1).astype(jnp.int32)
    diff_flat = difficult_table.reshape(-1)

    scaled_table = _scale_table(exercise_table, difficult_table)
    q_flat, pid_flat = _sc_gather_combine(
        e_flat, c_flat, scaled_table, concept_table, diff_flat)

    w1t = W[:, :D].T
    w2t = W[:, D:].T
    rows = a_table @ w2t + b[None, :]
    row0 = rows[0:1, :]
    diff = rows[1:2, :] - row0
    respf = response_seq.reshape(-1, 1).astype(jnp.float32)

    qa_flat = _tc_linear(q_flat, respf, w1t, row0, diff)

    q = q_flat.reshape(B, S, D)
    qa = qa_flat.reshape(B, S, D)
    pid = pid_flat.reshape(B, S, 1)
    return (q, qa, pid)
